# baseline (speedup 1.0000x reference)
import jax
import jax.numpy as jnp
from jax.experimental import pallas as pl
from jax.experimental.pallas import tpu as pltpu

_BM = 8192


def _add_body(x_ref, tab_ref, idx_ref, o_ref, pos_ref):
    reps, L = pos_ref.shape

    @pl.when(pl.program_id(0) == 0)
    def _gather():
        idx = idx_ref[...]
        tab = tab_ref[...]
        r = jax.lax.shift_right_logical(idx, (L - 1).bit_length())
        c = jnp.bitwise_and(idx, L - 1)
        acc = jnp.zeros((reps, L), jnp.float32)
        for r0 in range(reps):
            row = jax.lax.broadcast_in_dim(tab[r0, :], (reps, L), (1,))
            g = jnp.take_along_axis(row, c, axis=1)
            acc = jnp.where(r == r0, g, acc)
        pos_ref[...] = acc

    p = jnp.tile(pos_ref[...], (_BM // reps, 1))
    o_ref[...] = x_ref[...] + p


def kernel(inputs, pos_table, positions):
    B, S, _ = inputs.shape
    R = B * S // 128
    reps = S // 128
    x2 = inputs.reshape(R, 128)
    tab_tile = pos_table.reshape(reps, 128)
    idx_tile = positions.astype(jnp.int32).reshape(reps, 128)
    out = pl.pallas_call(
        _add_body,
        grid=(R // _BM,),
        in_specs=[
            pl.BlockSpec((_BM, 128), lambda i: (i, 0)),
            pl.BlockSpec((reps, 128), lambda i: (0, 0)),
            pl.BlockSpec((reps, 128), lambda i: (0, 0)),
        ],
        out_specs=pl.BlockSpec((_BM, 128), lambda i: (i, 0)),
        out_shape=jax.ShapeDtypeStruct((R, 128), jnp.float32),
        scratch_shapes=[pltpu.VMEM((reps, 128), jnp.float32)],
    )(x2, tab_tile, idx_tile)
    return out.reshape(B, S, 1)


# speedup vs baseline: 1.0229x; 1.0229x over previous
import jax
import jax.numpy as jnp
from jax.experimental import pallas as pl
from jax.experimental.pallas import tpu as pltpu

_BM = 16384


def _add_body(x_ref, tab_ref, idx_ref, o_ref, pos_ref):
    reps, L = pos_ref.shape

    @pl.when(pl.program_id(0) == 0)
    def _gather():
        idx = idx_ref[...]
        tab = tab_ref[...]
        r = jax.lax.shift_right_logical(idx, (L - 1).bit_length())
        c = jnp.bitwise_and(idx, L - 1)
        acc = jnp.zeros((reps, L), jnp.float32)
        for r0 in range(reps):
            row = jax.lax.broadcast_in_dim(tab[r0, :], (reps, L), (1,))
            g = jnp.take_along_axis(row, c, axis=1)
            acc = jnp.where(r == r0, g, acc)
        pos_ref[...] = acc

    p = jnp.tile(pos_ref[...], (_BM // reps, 1))
    o_ref[...] = x_ref[...] + p


def kernel(inputs, pos_table, positions):
    B, S, _ = inputs.shape
    R = B * S // 128
    reps = S // 128
    x2 = inputs.reshape(R, 128)
    tab_tile = pos_table.reshape(reps, 128)
    idx_tile = positions.astype(jnp.int32).reshape(reps, 128)
    out = pl.pallas_call(
        _add_body,
        grid=(R // _BM,),
        in_specs=[
            pl.BlockSpec((_BM, 128), lambda i: (i, 0)),
            pl.BlockSpec((reps, 128), lambda i: (0, 0)),
            pl.BlockSpec((reps, 128), lambda i: (0, 0)),
        ],
        out_specs=pl.BlockSpec((_BM, 128), lambda i: (i, 0)),
        out_shape=jax.ShapeDtypeStruct((R, 128), jnp.float32),
        scratch_shapes=[pltpu.VMEM((reps, 128), jnp.float32)],
    )(x2, tab_tile, idx_tile)
    return out.reshape(B, S, 1)
